# baseline (speedup 1.0000x reference)
import functools

import jax
import jax.numpy as jnp
from jax import lax
from jax.experimental import pallas as pl
from jax.experimental.pallas import tpu as pltpu
from jax.experimental.pallas import tpu_sc as plsc

_L = 16


@functools.lru_cache(maxsize=None)
def _build(B, V, d, C):
    NC, NS = 2, 16
    NW = NC * NS
    b_per_w = B // NW
    NCH = b_per_w // C
    D = d * d

    mesh = plsc.VectorSubcoreMesh(core_axis_name="c", subcore_axis_name="s")

    @functools.partial(
        pl.kernel,
        out_type=jax.ShapeDtypeStruct((d, B * d), jnp.float32),
        mesh=mesh,
        compiler_params=pltpu.CompilerParams(needs_layout_passes=False),
        scratch_types=[
            pltpu.VMEM((b_per_w,), jnp.int32),
            pltpu.VMEM((2, C, D), jnp.float32),
            pltpu.VMEM((d, C * d), jnp.float32),
        ],
    )
    def k(x_hbm, t_hbm, o_hbm, iv, bb, ww):
        wid = lax.axis_index("s") * NC + lax.axis_index("c")
        base = wid * b_per_w
        pltpu.sync_copy(x_hbm.at[pl.ds(base, b_per_w)], iv)

        def outer(g, carry):
            pltpu.sync_copy(t_hbm.at[iv.at[pl.ds(g * C, C)]], bb.at[0])

            def body(c, carry2):
                si = jnp.zeros((_L,), jnp.int32)
                ri = jnp.full((_L,), c, jnp.int32)
                lanes = lax.iota(jnp.int32, _L)
                for jj in range(d):
                    for h in range(d // _L):
                        ci = (lanes + h * _L) * d + jj
                        ww[jj, pl.ds(c * d + h * _L, _L)] = plsc.load_gather(
                            bb, [si, ri, ci])
                return carry2

            lax.fori_loop(0, C, body, 0)
            pltpu.sync_copy(ww, o_hbm.at[:, pl.ds((base + g * C) * d, C * d)])
            return carry

        lax.fori_loop(0, NCH, outer, 0)

    return k


def kernel(x, T1, T2):
    B = x.shape[0]
    V, d1 = T1.shape[0], T1.shape[1]
    d2 = T2.shape[1]
    t1 = T1.reshape(V, d1 * d1)
    t2 = T2.reshape(V, d2 * d2)
    xi = x.astype(jnp.int32)
    o1t = _build(B, V, d1, 32)(xi, t1)
    o2t = _build(B, V, d2, 32)(xi, t2)
    return o1t.T, o2t.T


# speedup vs baseline: 1.2394x; 1.2394x over previous
import functools

import jax
import jax.numpy as jnp
from jax import lax
from jax.experimental import pallas as pl
from jax.experimental.pallas import tpu as pltpu
from jax.experimental.pallas import tpu_sc as plsc

_L = 16


@functools.lru_cache(maxsize=None)
def _build(B, V, d, C):
    NC, NS = 2, 16
    NW = NC * NS
    b_per_w = B // NW
    NCH = b_per_w // C
    D = d * d
    PITCH = C * d + 1

    mesh = plsc.VectorSubcoreMesh(core_axis_name="c", subcore_axis_name="s")

    @functools.partial(
        pl.kernel,
        out_type=jax.ShapeDtypeStruct((d, B * d), jnp.float32),
        mesh=mesh,
        compiler_params=pltpu.CompilerParams(needs_layout_passes=False),
        scratch_types=[
            pltpu.VMEM((b_per_w,), jnp.int32),
            pltpu.VMEM((2, C, D), jnp.float32),
            pltpu.VMEM((d, PITCH), jnp.float32),
            pltpu.SemaphoreType.DMA((2,)),
        ],
    )
    def k(x_hbm, t_hbm, o_hbm, iv, bb, ww, sem):
        wid = lax.axis_index("s") * NC + lax.axis_index("c")
        base = wid * b_per_w
        pltpu.sync_copy(x_hbm.at[pl.ds(base, b_per_w)], iv)

        def start(g, slot):
            pltpu.async_copy(
                t_hbm.at[iv.at[pl.ds(g * C, C)]], bb.at[slot], sem.at[slot])

        def finish(g, slot):
            pltpu.make_async_copy(
                t_hbm.at[iv.at[pl.ds(g * C, C)]], bb.at[slot],
                sem.at[slot]).wait()

            def body(c, carry):
                lanes = lax.iota(jnp.int32, _L)
                for ii in range(d):
                    col = jnp.full((_L,), c * d + ii, jnp.int32)
                    for h in range(d // _L):
                        v = bb[slot, c, pl.ds(ii * d + h * _L, _L)]
                        plsc.store_scatter(ww, [lanes + h * _L, col], v)
                return carry

            lax.fori_loop(0, C, body, 0)
            pltpu.sync_copy(ww.at[:, pl.ds(0, C * d)],
                            o_hbm.at[:, pl.ds((base + g * C) * d, C * d)])

        start(0, 0)
        start(1, 1)

        def ring(h, carry):
            g0 = 2 * h
            for b in range(2):
                g = g0 + b
                finish(g, b)
                pl.when(g + 2 < NCH)(lambda: start(g + 2, b))
            return carry

        lax.fori_loop(0, NCH // 2, ring, 0)

    return k


def kernel(x, T1, T2):
    B = x.shape[0]
    V, d1 = T1.shape[0], T1.shape[1]
    d2 = T2.shape[1]
    t1 = T1.reshape(V, d1 * d1)
    t2 = T2.reshape(V, d2 * d2)
    xi = x.astype(jnp.int32)
    o1t = _build(B, V, d1, 32)(xi, t1)
    o2t = _build(B, V, d2, 32)(xi, t2)
    return o1t.T, o2t.T
